# baseline (speedup 1.0000x reference)
import functools

import jax
import jax.numpy as jnp
from jax import lax
from jax.experimental import pallas as pl
from jax.experimental.pallas import tpu as pltpu
from jax.experimental.pallas import tpu_sc as plsc

NC = 2
NS = 16
NW = NC * NS
LANES = 16
CH = 1024
CHB = 256
IDX_G = CH // 128
IDX_GB = CHB // 128

_SC_PARAMS = dict(
    compiler_params=pltpu.CompilerParams(use_tc_tiling_on_sc=False),
)


def _zero_rows(buf, rows, cols):
  zero = jnp.zeros((LANES,), jnp.float32)
  nv = cols // LANES

  def st(e, _):
    for v in range(nv):
      buf[e, pl.ds(v * LANES, LANES)] = zero
    return 0

  lax.fori_loop(0, rows, st, 0, unroll=8)


def _zero_acc_slice(buf, acc, r0, rpt, rows):
  for q in range(rpt // rows):
    pltpu.sync_copy(buf, acc.at[pl.ds(r0 + q * rows, rows)])
  rem = rpt % rows
  if rem:
    pltpu.sync_copy(buf.at[pl.ds(0, rem)],
                    acc.at[pl.ds(r0 + (rpt // rows) * rows, rem)])


def _mesh():
  return plsc.VectorSubcoreMesh(
      core_axis_name="c", subcore_axis_name="s", num_cores=NC,
      num_subcores=NS)


def _make_edge_weight_kernel(n_pad, e_pad):
  nchunk = e_pad // NW // CH
  rpt = n_pad // NS

  @functools.partial(
      pl.kernel,
      out_type=(
          jax.ShapeDtypeStruct((e_pad, 16), jnp.float32),
          jax.ShapeDtypeStruct((NC, n_pad, 16), jnp.float32),
      ),
      mesh=_mesh(),
      scratch_types=[
          pltpu.VMEM((IDX_G, 128), jnp.int32),
          pltpu.VMEM((IDX_G, 128), jnp.int32),
          pltpu.VMEM((CH, 16), jnp.float32),
          pltpu.VMEM((CH, 16), jnp.float32),
          pltpu.VMEM((CH, 16), jnp.float32),
          pltpu.SemaphoreType.DMA,
          pltpu.VMEM_SHARED((n_pad, 16), jnp.float32),
      ],
      **_SC_PARAMS,
  )
  def kern(src_hbm, dst_hbm, asrc_hbm, adst_hbm, w_hbm, den_hbm,
           idx_s, idx_d, a_s, a_d, w_r, sem, den_acc):
    c = lax.axis_index("c")
    s = lax.axis_index("s")
    wid = s * NC + c
    r0 = pl.multiple_of(s * rpt, 64)

    _zero_rows(w_r, CH, 16)
    _zero_acc_slice(w_r, den_acc, r0, rpt, CH)
    plsc.subcore_barrier()

    def chunk(ci, _):
      base = pl.multiple_of((wid * nchunk + ci) * CH, CH)
      row0 = pl.multiple_of(base // 128, IDX_G)
      pltpu.sync_copy(src_hbm.at[pl.ds(row0, IDX_G)], idx_s)
      pltpu.sync_copy(dst_hbm.at[pl.ds(row0, IDX_G)], idx_d)
      cps = []
      for j in range(IDX_G):
        cps.append(pltpu.async_copy(
            asrc_hbm.at[idx_s.at[j]], a_s.at[pl.ds(j * 128, 128)], sem))
        cps.append(pltpu.async_copy(
            adst_hbm.at[idx_d.at[j]], a_d.at[pl.ds(j * 128, 128)], sem))
      for cp in cps:
        cp.wait()

      def estep(e, _):
        x = a_s[e, pl.ds(0, 16)] + a_d[e, pl.ds(0, 16)]
        w_r[e, pl.ds(0, 16)] = jnp.exp(jnp.maximum(x, 0.2 * x))
        return 0

      lax.fori_loop(0, CH, estep, 0, unroll=8)

      pltpu.sync_copy(w_r, w_hbm.at[pl.ds(base, CH)])
      for j in range(IDX_G):
        pltpu.sync_copy(w_r.at[pl.ds(j * 128, 128)],
                        den_acc.at[idx_d.at[j]], add=True)
      return 0

    lax.fori_loop(0, nchunk, chunk, 0)
    plsc.subcore_barrier()
    pltpu.sync_copy(den_acc.at[pl.ds(r0, rpt)],
                    den_hbm.at[c, pl.ds(r0, rpt)])

  return kern


def _make_aggregate1_kernel(n_pad, e_pad):
  CHA = 128
  nchunk = e_pad // NS // CHA
  rpt = n_pad // NS
  MB = 20
  nmb = nchunk // MB
  npair = MB // 2

  @functools.partial(
      pl.kernel,
      out_type=jax.ShapeDtypeStruct((16, n_pad, 32), jnp.float32),
      mesh=_mesh(),
      scratch_types=[
          pltpu.VMEM((MB, 128), jnp.int32),
          pltpu.VMEM((MB, 128), jnp.int32),
          pltpu.VMEM((CHA, 16), jnp.float32),
          pltpu.VMEM((CHA, 16), jnp.float32),
          pltpu.VMEM((CHA, 32), jnp.float32),
          pltpu.VMEM((CHA, 32), jnp.float32),
          pltpu.VMEM((CHA, 32), jnp.float32),
          pltpu.VMEM((CHA, 32), jnp.float32),
          pltpu.SemaphoreType.DMA,
          pltpu.SemaphoreType.DMA,
          pltpu.SemaphoreType.DMA,
          pltpu.SemaphoreType.DMA,
          pltpu.VMEM_SHARED((n_pad, 32), jnp.float32),
      ],
      **_SC_PARAMS,
  )
  def kern(src_hbm, dst_hbm, ht_hbm, w_hbm, outp_hbm,
           idx_s, idx_d, w0, w1, h0, h1, m0, m1,
           gsem0, gsem1, ssem0, ssem1, out_acc):
    c = lax.axis_index("c")
    s = lax.axis_index("s")
    r0 = pl.multiple_of(s * rpt, 64)
    wb = (w0, w1)
    hb = (h0, h1)
    mb_ = (m0, m1)
    gsems = (gsem0, gsem1)
    ssems = (ssem0, ssem1)

    def one_pass(p):
      head = p // 2
      _zero_rows(m0, CHA, 32)
      zcps = []
      for q in range(rpt // CHA):
        zcps.append(pltpu.async_copy(
            m0, out_acc.at[pl.ds(r0 + q * CHA, CHA)], gsem0))
      rem = rpt % CHA
      if rem:
        zcps.append(pltpu.async_copy(
            m0.at[pl.ds(0, rem)],
            out_acc.at[pl.ds(r0 + (rpt // CHA) * CHA, rem)], gsem0))
      for cp in zcps:
        cp.wait()
      plsc.subcore_barrier()

      def mb_body(m, _):
        mbase = pl.multiple_of((s * nchunk + m * MB) * CHA, CHA)
        mrow0 = pl.multiple_of(mbase // 128, 1)
        pltpu.sync_copy(src_hbm.at[pl.ds(mrow0, MB)], idx_s)
        pltpu.sync_copy(dst_hbm.at[pl.ds(mrow0, MB)], idx_d)

        def fire(k, b):
          cbase = pl.multiple_of(mbase + k * CHA, CHA)
          pltpu.async_copy(w_hbm.at[pl.ds(cbase, CHA)], wb[b], gsems[b])
          pltpu.async_copy(ht_hbm.at[p].at[idx_s.at[k]], hb[b], gsems[b])

        def drain_g(b):
          pltpu.make_async_copy(
              w_hbm.at[pl.ds(mbase, CHA)], wb[b], gsems[b]).wait()
          pltpu.make_async_copy(
              ht_hbm.at[p].at[idx_s.at[0]], hb[b], gsems[b]).wait()

        def drain_s(b):
          pltpu.make_async_copy(
              mb_[b], out_acc.at[idx_d.at[0]], ssems[b]).wait()

        fire(0, 0)
        fire(1, 1)

        def pair(q, _):
          for b in range(2):
            k = 2 * q + b
            drain_g(b)

            @pl.when(q >= 1)
            def _():
              drain_s(b)
            wbuf = wb[b]
            hrows = hb[b]
            msg = mb_[b]

            def estep(e, _):
              wv = jnp.full(
                  (LANES,), wbuf[e, pl.ds(0, 16)][head], jnp.float32)
              msg[e, pl.ds(0, 16)] = hrows[e, pl.ds(0, 16)] * wv
              msg[e, pl.ds(16, 16)] = hrows[e, pl.ds(16, 16)] * wv
              return 0

            lax.fori_loop(0, CHA, estep, 0, unroll=8)
            pltpu.async_copy(msg, out_acc.at[idx_d.at[k]], ssems[b],
                             add=True)

            @pl.when(q < npair - 1)
            def _():
              fire(k + 2, b)
          return 0

        lax.fori_loop(0, npair, pair, 0)
        drain_s(0)
        drain_s(1)
        return 0

      lax.fori_loop(0, nmb, mb_body, 0)
      plsc.subcore_barrier()
      pltpu.sync_copy(out_acc.at[pl.ds(r0, rpt)],
                      outp_hbm.at[p, pl.ds(r0, rpt)])

    for pp in range(8):
      for cc in range(2):
        @pl.when(c == cc)
        def _():
          one_pass(cc * 8 + pp)

  return kern


def _make_layer2_kernel(n_pad, e_pad):
  nchunk = e_pad // NW // CHB
  rpt = n_pad // NS
  MB = 10
  nmb = nchunk // MB
  npair = MB // 2

  @functools.partial(
      pl.kernel,
      out_type=jax.ShapeDtypeStruct((NC, n_pad, 32), jnp.float32),
      mesh=_mesh(),
      scratch_types=[
          pltpu.VMEM((MB * IDX_GB, 128), jnp.int32),
          pltpu.VMEM((MB * IDX_GB, 128), jnp.int32),
          pltpu.VMEM((CHB, 16), jnp.float32),
          pltpu.VMEM((CHB, 16), jnp.float32),
          pltpu.VMEM((CHB, 32), jnp.float32),
          pltpu.VMEM((CHB, 32), jnp.float32),
          pltpu.SemaphoreType.DMA,
          pltpu.SemaphoreType.DMA,
          pltpu.VMEM_SHARED((n_pad, 32), jnp.float32),
      ],
      **_SC_PARAMS,
  )
  def kern(src_hbm, dst_hbm, gtab_hbm, atab_hbm, outp_hbm,
           idx_s, idx_d, a0, a1, g0, g1, sem0, sem1, out_acc):
    c = lax.axis_index("c")
    s = lax.axis_index("s")
    wid = s * NC + c
    r0 = pl.multiple_of(s * rpt, 64)
    ab = (a0, a1)
    gb = (g0, g1)
    sems = (sem0, sem1)

    _zero_rows(g0, CHB, 32)
    _zero_acc_slice(g0, out_acc, r0, rpt, CHB)
    plsc.subcore_barrier()

    def mb_body(m, _):
      mbase = pl.multiple_of((wid * nchunk + m * MB) * CHB, CHB)
      mrow0 = pl.multiple_of(mbase // 128, IDX_GB)
      pltpu.sync_copy(src_hbm.at[pl.ds(mrow0, MB * IDX_GB)], idx_s)
      pltpu.sync_copy(dst_hbm.at[pl.ds(mrow0, MB * IDX_GB)], idx_d)

      def fire(k, b):
        for j in range(IDX_GB):
          pltpu.async_copy(
              gtab_hbm.at[idx_s.at[k * IDX_GB + j]],
              gb[b].at[pl.ds(j * 128, 128)], sems[b])
          pltpu.async_copy(
              atab_hbm.at[idx_d.at[k * IDX_GB + j]],
              ab[b].at[pl.ds(j * 128, 128)], sems[b])

      def drain(b):
        for j in range(IDX_GB):
          pltpu.make_async_copy(
              gtab_hbm.at[idx_s.at[j]],
              gb[b].at[pl.ds(j * 128, 128)], sems[b]).wait()
          pltpu.make_async_copy(
              atab_hbm.at[idx_d.at[j]],
              ab[b].at[pl.ds(j * 128, 128)], sems[b]).wait()

      fire(0, 0)
      fire(1, 1)

      def pair(q, _):
        for b in range(2):
          k = 2 * q + b
          drain(b)
          arows = ab[b]
          grows = gb[b]

          def estep(e, _):
            g0v = grows[e, pl.ds(0, 16)]
            g1v = grows[e, pl.ds(16, 16)]
            xv = (jnp.full((LANES,), g1v[2], jnp.float32)
                  + jnp.full((LANES,), arows[e, pl.ds(0, 16)][0],
                             jnp.float32))
            wv = jnp.exp(jnp.maximum(xv, 0.2 * xv))
            grows[e, pl.ds(0, 16)] = g0v * wv
            grows[e, pl.ds(16, 16)] = g1v * wv
            return 0

          lax.fori_loop(0, CHB, estep, 0, unroll=4)
          for j in range(IDX_GB):
            pltpu.sync_copy(grows.at[pl.ds(j * 128, 128)],
                            out_acc.at[idx_d.at[k * IDX_GB + j]], add=True)

          @pl.when(q < npair - 1)
          def _():
            fire(k + 2, b)
        return 0

      lax.fori_loop(0, npair, pair, 0)
      return 0

    lax.fori_loop(0, nmb, mb_body, 0)
    plsc.subcore_barrier()
    pltpu.sync_copy(out_acc.at[pl.ds(r0, rpt)],
                    outp_hbm.at[c, pl.ds(r0, rpt)])

  return kern


def _tc_project1(xp, W1, avs, avd, n_pad):
  nb = n_pad // 256

  def body(x_ref, w_ref, avs_ref, avd_ref, h_ref, ht_ref, as_ref, ad_ref,
           ws_ref):
    h = jnp.dot(x_ref[...], w_ref[...], preferred_element_type=jnp.float32)
    h_ref[...] = h
    ht_ref[...] = jnp.transpose(h.reshape(256, 16, 32), (1, 0, 2))
    a_s = jnp.dot(h, avs_ref[...], preferred_element_type=jnp.float32)
    a_d = jnp.dot(h, avd_ref[...], preferred_element_type=jnp.float32)
    as_ref[...] = a_s
    ad_ref[...] = a_d
    x2 = a_s + a_d
    ws_ref[...] = jnp.exp(jnp.maximum(x2, 0.2 * x2))

  return pl.pallas_call(
      body,
      grid=(nb,),
      in_specs=[
          pl.BlockSpec((256, 64), lambda i: (i, 0)),
          pl.BlockSpec((64, 512), lambda i: (0, 0)),
          pl.BlockSpec((512, 16), lambda i: (0, 0)),
          pl.BlockSpec((512, 16), lambda i: (0, 0)),
      ],
      out_specs=[
          pl.BlockSpec((256, 512), lambda i: (i, 0)),
          pl.BlockSpec((16, 256, 32), lambda i: (0, i, 0)),
          pl.BlockSpec((256, 16), lambda i: (i, 0)),
          pl.BlockSpec((256, 16), lambda i: (i, 0)),
          pl.BlockSpec((256, 16), lambda i: (i, 0)),
      ],
      out_shape=[
          jax.ShapeDtypeStruct((n_pad, 512), jnp.float32),
          jax.ShapeDtypeStruct((16, n_pad, 32), jnp.float32),
          jax.ShapeDtypeStruct((n_pad, 16), jnp.float32),
          jax.ShapeDtypeStruct((n_pad, 16), jnp.float32),
          jax.ShapeDtypeStruct((n_pad, 16), jnp.float32),
      ],
  )(xp, W1, avs, avd)


def _tc_epilogue1(h, outsc, wself, den_sum, e8, bias1, n_pad):
  nb = n_pad // 256

  def body(h_ref, o_ref, ws_ref, dn_ref, e8_ref, b_ref, out_ref):
    e8m = e8_ref[...]
    wrep = jnp.dot(ws_ref[...], e8m, preferred_element_type=jnp.float32)
    den = dn_ref[...] + ws_ref[...]
    drep = jnp.dot(den, e8m, preferred_element_type=jnp.float32)
    osc = jnp.transpose(o_ref[...], (1, 0, 2)).reshape(256, 512)
    num = osc + wrep * h_ref[...]
    v = num / drep + b_ref[...]
    out_ref[...] = jnp.where(v > 0, v, jnp.exp(jnp.minimum(v, 0.0)) - 1.0)

  return pl.pallas_call(
      body,
      grid=(nb,),
      in_specs=[
          pl.BlockSpec((256, 512), lambda i: (i, 0)),
          pl.BlockSpec((16, 256, 32), lambda i: (0, i, 0)),
          pl.BlockSpec((256, 16), lambda i: (i, 0)),
          pl.BlockSpec((256, 16), lambda i: (i, 0)),
          pl.BlockSpec((16, 512), lambda i: (0, 0)),
          pl.BlockSpec((1, 512), lambda i: (0, 0)),
      ],
      out_specs=pl.BlockSpec((256, 512), lambda i: (i, 0)),
      out_shape=jax.ShapeDtypeStruct((n_pad, 512), jnp.float32),
  )(h, outsc, wself, den_sum, e8, bias1)


def _tc_project2(h1, W48, onehot17, n_pad):
  nb = n_pad // 256

  def body(h_ref, w_ref, oh_ref, out_ref, gt_ref, at_ref):
    g = jnp.dot(
        h_ref[...], w_ref[...], preferred_element_type=jnp.float32
    ) + oh_ref[...]
    out_ref[...] = g
    gt_ref[...] = g[:, 0:32]
    col = lax.broadcasted_iota(jnp.int32, (256, 16), 1)
    at_ref[...] = jnp.where(col == 0, g[:, 32:33], 0.0)

  return pl.pallas_call(
      body,
      grid=(nb,),
      in_specs=[
          pl.BlockSpec((256, 512), lambda i: (i, 0)),
          pl.BlockSpec((512, 48), lambda i: (0, 0)),
          pl.BlockSpec((1, 48), lambda i: (0, 0)),
      ],
      out_specs=[
          pl.BlockSpec((256, 48), lambda i: (i, 0)),
          pl.BlockSpec((256, 32), lambda i: (i, 0)),
          pl.BlockSpec((256, 16), lambda i: (i, 0)),
      ],
      out_shape=[
          jax.ShapeDtypeStruct((n_pad, 48), jnp.float32),
          jax.ShapeDtypeStruct((n_pad, 32), jnp.float32),
          jax.ShapeDtypeStruct((n_pad, 16), jnp.float32),
      ],
  )(h1, W48, onehot17)


def _tc_epilogue2(g48, p0, p1, bias2p, n_pad):
  nb = n_pad // 256

  def body(g_ref, p0_ref, p1_ref, b_ref, out_ref):
    g = g_ref[...]
    g32 = g[:, 0:32]
    a2 = g[:, 18:19] + g[:, 32:33]
    ws2 = jnp.exp(jnp.maximum(a2, 0.2 * a2))
    num = p0_ref[...] + p1_ref[...] + ws2 * g32
    den = num[:, 17:18]
    logit = num / den + b_ref[...]
    col = lax.broadcasted_iota(jnp.int32, logit.shape, 1)
    masked = jnp.where(col < 17, logit, -1e30)
    mx = jnp.max(masked, axis=1, keepdims=True)
    lse = jnp.log(jnp.sum(jnp.exp(masked - mx), axis=1, keepdims=True)) + mx
    out_ref[...] = masked - lse

  return pl.pallas_call(
      body,
      grid=(nb,),
      in_specs=[
          pl.BlockSpec((256, 48), lambda i: (i, 0)),
          pl.BlockSpec((256, 32), lambda i: (i, 0)),
          pl.BlockSpec((256, 32), lambda i: (i, 0)),
          pl.BlockSpec((1, 32), lambda i: (0, 0)),
      ],
      out_specs=pl.BlockSpec((256, 32), lambda i: (i, 0)),
      out_shape=jax.ShapeDtypeStruct((n_pad, 32), jnp.float32),
  )(g48, p0, p1, bias2p)


def kernel(x, edge_index, W1, att_src1, att_dst1, bias1,
           W2, att_src2, att_dst2, bias2):
  n = x.shape[0]
  e = edge_index.shape[1]
  n_pad = ((n + 1 + 255) // 256) * 256
  e_pad = ((e + NW * CH - 1) // (NW * CH)) * (NW * CH)

  src = jnp.concatenate(
      [edge_index[0], jnp.full((e_pad - e,), n, jnp.int32)]
  ).reshape(e_pad // 128, 128)
  dst = jnp.concatenate(
      [edge_index[1], jnp.full((e_pad - e,), n, jnp.int32)]
  ).reshape(e_pad // 128, 128)
  xp = jnp.pad(x, ((0, n_pad - n), (0, 0)))

  eye8 = jnp.eye(8, dtype=jnp.float32)
  avs = jnp.pad((eye8[:, None, :] * att_src1[:, :, None]).reshape(512, 8),
                ((0, 0), (0, 8)))
  avd = jnp.pad((eye8[:, None, :] * att_dst1[:, :, None]).reshape(512, 8),
                ((0, 0), (0, 8)))
  e8 = jnp.repeat(jnp.pad(eye8, ((0, 8), (0, 0))), 64, axis=1)
  b1 = bias1.reshape(1, 512)

  h, h_t, asrc, adst, wself = _tc_project1(xp, W1, avs, avd, n_pad)
  w_rows, den_p = _make_edge_weight_kernel(n_pad, e_pad)(
      src, dst, asrc, adst)
  outp = _make_aggregate1_kernel(n_pad, e_pad)(src, dst, h_t, w_rows)
  den_sum = den_p[0] + den_p[1]
  h1 = _tc_epilogue1(h, outp, wself, den_sum, e8, b1, n_pad)

  gs = W2 @ att_src2.reshape(17)
  gd = W2 @ att_dst2.reshape(17)
  W48 = jnp.concatenate(
      [W2, jnp.zeros((512, 1), jnp.float32), gs[:, None],
       jnp.zeros((512, 13), jnp.float32), gd[:, None],
       jnp.zeros((512, 15), jnp.float32)], axis=1)
  onehot17 = (jnp.arange(48) == 17).astype(jnp.float32).reshape(1, 48)
  g48, gtab, atab = _tc_project2(h1, W48, onehot17, n_pad)
  out2p = _make_layer2_kernel(n_pad, e_pad)(src, dst, gtab, atab)
  b2 = jnp.pad(bias2, (0, 15)).reshape(1, 32)
  lsm = _tc_epilogue2(g48, out2p[0], out2p[1], b2, n_pad)
  return lsm[:n, :17]


# speedup vs baseline: 1.1050x; 1.1050x over previous
import functools

import jax
import jax.numpy as jnp
from jax import lax
from jax.experimental import pallas as pl
from jax.experimental.pallas import tpu as pltpu
from jax.experimental.pallas import tpu_sc as plsc

NC = 2
NS = 16
NW = NC * NS
LANES = 16
CH = 1024
CHB = 256
IDX_G = CH // 128
IDX_GB = CHB // 128

_SC_PARAMS = dict(
    compiler_params=pltpu.CompilerParams(use_tc_tiling_on_sc=False),
)


def _zero_rows(buf, rows, cols):
  zero = jnp.zeros((LANES,), jnp.float32)
  nv = cols // LANES

  def st(e, _):
    for v in range(nv):
      buf[e, pl.ds(v * LANES, LANES)] = zero
    return 0

  lax.fori_loop(0, rows, st, 0, unroll=8)


def _zero_acc_slice(buf, acc, r0, rpt, rows):
  for q in range(rpt // rows):
    pltpu.sync_copy(buf, acc.at[pl.ds(r0 + q * rows, rows)])
  rem = rpt % rows
  if rem:
    pltpu.sync_copy(buf.at[pl.ds(0, rem)],
                    acc.at[pl.ds(r0 + (rpt // rows) * rows, rem)])


def _mesh():
  return plsc.VectorSubcoreMesh(
      core_axis_name="c", subcore_axis_name="s", num_cores=NC,
      num_subcores=NS)


def _make_edge_weight_kernel(n_pad, e_pad):
  CHA = 512
  GA = CHA // 128
  nchunk = e_pad // NW // CHA
  rpt = n_pad // NS
  MB = 10
  nmb = nchunk // MB
  npair = MB // 2

  @functools.partial(
      pl.kernel,
      out_type=(
          jax.ShapeDtypeStruct((e_pad, 16), jnp.float32),
          jax.ShapeDtypeStruct((NC, n_pad, 16), jnp.float32),
      ),
      mesh=_mesh(),
      scratch_types=[
          pltpu.VMEM((MB * GA, 128), jnp.int32),
          pltpu.VMEM((MB * GA, 128), jnp.int32),
          pltpu.VMEM((CHA, 16), jnp.float32),
          pltpu.VMEM((CHA, 16), jnp.float32),
          pltpu.VMEM((CHA, 16), jnp.float32),
          pltpu.VMEM((CHA, 16), jnp.float32),
          pltpu.SemaphoreType.DMA,
          pltpu.SemaphoreType.DMA,
          pltpu.VMEM_SHARED((n_pad, 16), jnp.float32),
      ],
      **_SC_PARAMS,
  )
  def kern(src_hbm, dst_hbm, asrc_hbm, adst_hbm, w_hbm, den_hbm,
           idx_s, idx_d, s0, s1, d0, d1, sem0, sem1, den_acc):
    c = lax.axis_index("c")
    s = lax.axis_index("s")
    wid = s * NC + c
    r0 = pl.multiple_of(s * rpt, 64)
    sb = (s0, s1)
    db = (d0, d1)
    sems = (sem0, sem1)

    _zero_rows(s0, CHA, 16)
    _zero_acc_slice(s0, den_acc, r0, rpt, CHA)
    plsc.subcore_barrier()

    def mb_body(m, _):
      mbase = pl.multiple_of((wid * nchunk + m * MB) * CHA, CHA)
      mrow0 = pl.multiple_of(mbase // 128, GA)
      pltpu.sync_copy(src_hbm.at[pl.ds(mrow0, MB * GA)], idx_s)
      pltpu.sync_copy(dst_hbm.at[pl.ds(mrow0, MB * GA)], idx_d)

      def fire(k, b):
        for j in range(GA):
          pltpu.async_copy(
              asrc_hbm.at[idx_s.at[k * GA + j]],
              sb[b].at[pl.ds(j * 128, 128)], sems[b])
          pltpu.async_copy(
              adst_hbm.at[idx_d.at[k * GA + j]],
              db[b].at[pl.ds(j * 128, 128)], sems[b])

      def drain(b):
        for j in range(GA):
          pltpu.make_async_copy(
              asrc_hbm.at[idx_s.at[j]],
              sb[b].at[pl.ds(j * 128, 128)], sems[b]).wait()
          pltpu.make_async_copy(
              adst_hbm.at[idx_d.at[j]],
              db[b].at[pl.ds(j * 128, 128)], sems[b]).wait()

      fire(0, 0)
      fire(1, 1)

      def pair(q, _):
        for b in range(2):
          k = 2 * q + b
          drain(b)
          a_s = sb[b]
          a_d = db[b]

          def estep(e, _):
            x = a_s[e, pl.ds(0, 16)] + a_d[e, pl.ds(0, 16)]
            a_s[e, pl.ds(0, 16)] = jnp.exp(jnp.maximum(x, 0.2 * x))
            return 0

          lax.fori_loop(0, CHA, estep, 0, unroll=8)
          cbase = pl.multiple_of(mbase + k * CHA, CHA)
          pltpu.sync_copy(a_s, w_hbm.at[pl.ds(cbase, CHA)])
          for j in range(GA):
            pltpu.sync_copy(a_s.at[pl.ds(j * 128, 128)],
                            den_acc.at[idx_d.at[k * GA + j]], add=True)

          @pl.when(q < npair - 1)
          def _():
            fire(k + 2, b)
        return 0

      lax.fori_loop(0, npair, pair, 0)
      return 0

    lax.fori_loop(0, nmb, mb_body, 0)
    plsc.subcore_barrier()
    pltpu.sync_copy(den_acc.at[pl.ds(r0, rpt)],
                    den_hbm.at[c, pl.ds(r0, rpt)])

  return kern


def _make_aggregate1_kernel(n_pad, e_pad):
  nchunk = e_pad // NS // CHB
  rpt = n_pad // NS
  MB = 10
  nmb = nchunk // MB
  npair = MB // 2

  @functools.partial(
      pl.kernel,
      out_type=jax.ShapeDtypeStruct((16, n_pad, 32), jnp.float32),
      mesh=_mesh(),
      scratch_types=[
          pltpu.VMEM((MB * IDX_GB, 128), jnp.int32),
          pltpu.VMEM((MB * IDX_GB, 128), jnp.int32),
          pltpu.VMEM((CHB, 16), jnp.float32),
          pltpu.VMEM((CHB, 16), jnp.float32),
          pltpu.VMEM((CHB, 32), jnp.float32),
          pltpu.VMEM((CHB, 32), jnp.float32),
          pltpu.SemaphoreType.DMA,
          pltpu.SemaphoreType.DMA,
          pltpu.VMEM_SHARED((n_pad, 32), jnp.float32),
      ],
      **_SC_PARAMS,
  )
  def kern(src_hbm, dst_hbm, ht_hbm, w_hbm, outp_hbm,
           idx_s, idx_d, w0, w1, h0, h1, sem0, sem1, out_acc):
    c = lax.axis_index("c")
    s = lax.axis_index("s")
    r0 = pl.multiple_of(s * rpt, 64)
    wb = (w0, w1)
    hb = (h0, h1)
    sems = (sem0, sem1)

    def one_pass(p):
      head = p // 2
      _zero_rows(h0, CHB, 32)
      _zero_acc_slice(h0, out_acc, r0, rpt, CHB)
      plsc.subcore_barrier()

      def mb_body(m, _):
        mbase = pl.multiple_of((s * nchunk + m * MB) * CHB, CHB)
        mrow0 = pl.multiple_of(mbase // 128, IDX_GB)
        pltpu.sync_copy(src_hbm.at[pl.ds(mrow0, MB * IDX_GB)], idx_s)
        pltpu.sync_copy(dst_hbm.at[pl.ds(mrow0, MB * IDX_GB)], idx_d)

        def fire(k, b):
          cbase = pl.multiple_of(mbase + k * CHB, CHB)
          pltpu.async_copy(w_hbm.at[pl.ds(cbase, CHB)], wb[b], sems[b])
          for j in range(IDX_GB):
            pltpu.async_copy(
                ht_hbm.at[p].at[idx_s.at[k * IDX_GB + j]],
                hb[b].at[pl.ds(j * 128, 128)], sems[b])

        def drain(b):
          pltpu.make_async_copy(
              w_hbm.at[pl.ds(mbase, CHB)], wb[b], sems[b]).wait()
          for j in range(IDX_GB):
            pltpu.make_async_copy(
                ht_hbm.at[p].at[idx_s.at[j]],
                hb[b].at[pl.ds(j * 128, 128)], sems[b]).wait()

        fire(0, 0)
        fire(1, 1)

        def pair(q, _):
          for b in range(2):
            k = 2 * q + b
            drain(b)
            wbuf = wb[b]
            hrows = hb[b]

            def estep(e, _):
              wv = jnp.full(
                  (LANES,), wbuf[e, pl.ds(0, 16)][head], jnp.float32)
              hrows[e, pl.ds(0, 16)] = hrows[e, pl.ds(0, 16)] * wv
              hrows[e, pl.ds(16, 16)] = hrows[e, pl.ds(16, 16)] * wv
              return 0

            lax.fori_loop(0, CHB, estep, 0, unroll=8)
            for j in range(IDX_GB):
              pltpu.sync_copy(hrows.at[pl.ds(j * 128, 128)],
                              out_acc.at[idx_d.at[k * IDX_GB + j]], add=True)

            @pl.when(q < npair - 1)
            def _():
              fire(k + 2, b)
          return 0

        lax.fori_loop(0, npair, pair, 0)
        return 0

      lax.fori_loop(0, nmb, mb_body, 0)
      plsc.subcore_barrier()
      pltpu.sync_copy(out_acc.at[pl.ds(r0, rpt)],
                      outp_hbm.at[p, pl.ds(r0, rpt)])

    for pp in range(8):
      for cc in range(2):
        @pl.when(c == cc)
        def _():
          one_pass(cc * 8 + pp)

  return kern


def _make_layer2_kernel(n_pad, e_pad):
  nchunk = e_pad // NW // CHB
  rpt = n_pad // NS
  MB = 10
  nmb = nchunk // MB
  npair = MB // 2

  @functools.partial(
      pl.kernel,
      out_type=jax.ShapeDtypeStruct((NC, n_pad, 32), jnp.float32),
      mesh=_mesh(),
      scratch_types=[
          pltpu.VMEM((MB * IDX_GB, 128), jnp.int32),
          pltpu.VMEM((MB * IDX_GB, 128), jnp.int32),
          pltpu.VMEM((CHB, 16), jnp.float32),
          pltpu.VMEM((CHB, 16), jnp.float32),
          pltpu.VMEM((CHB, 32), jnp.float32),
          pltpu.VMEM((CHB, 32), jnp.float32),
          pltpu.SemaphoreType.DMA,
          pltpu.SemaphoreType.DMA,
          pltpu.VMEM_SHARED((n_pad, 32), jnp.float32),
      ],
      **_SC_PARAMS,
  )
  def kern(src_hbm, dst_hbm, gtab_hbm, atab_hbm, outp_hbm,
           idx_s, idx_d, a0, a1, g0, g1, sem0, sem1, out_acc):
    c = lax.axis_index("c")
    s = lax.axis_index("s")
    wid = s * NC + c
    r0 = pl.multiple_of(s * rpt, 64)
    ab = (a0, a1)
    gb = (g0, g1)
    sems = (sem0, sem1)

    _zero_rows(g0, CHB, 32)
    _zero_acc_slice(g0, out_acc, r0, rpt, CHB)
    plsc.subcore_barrier()

    def mb_body(m, _):
      mbase = pl.multiple_of((wid * nchunk + m * MB) * CHB, CHB)
      mrow0 = pl.multiple_of(mbase // 128, IDX_GB)
      pltpu.sync_copy(src_hbm.at[pl.ds(mrow0, MB * IDX_GB)], idx_s)
      pltpu.sync_copy(dst_hbm.at[pl.ds(mrow0, MB * IDX_GB)], idx_d)

      def fire(k, b):
        for j in range(IDX_GB):
          pltpu.async_copy(
              gtab_hbm.at[idx_s.at[k * IDX_GB + j]],
              gb[b].at[pl.ds(j * 128, 128)], sems[b])
          pltpu.async_copy(
              atab_hbm.at[idx_d.at[k * IDX_GB + j]],
              ab[b].at[pl.ds(j * 128, 128)], sems[b])

      def drain(b):
        for j in range(IDX_GB):
          pltpu.make_async_copy(
              gtab_hbm.at[idx_s.at[j]],
              gb[b].at[pl.ds(j * 128, 128)], sems[b]).wait()
          pltpu.make_async_copy(
              atab_hbm.at[idx_d.at[j]],
              ab[b].at[pl.ds(j * 128, 128)], sems[b]).wait()

      fire(0, 0)
      fire(1, 1)

      def pair(q, _):
        for b in range(2):
          k = 2 * q + b
          drain(b)
          arows = ab[b]
          grows = gb[b]

          def estep(e, _):
            g0v = grows[e, pl.ds(0, 16)]
            g1v = grows[e, pl.ds(16, 16)]
            xv = (jnp.full((LANES,), g1v[2], jnp.float32)
                  + jnp.full((LANES,), arows[e, pl.ds(0, 16)][0],
                             jnp.float32))
            wv = jnp.exp(jnp.maximum(xv, 0.2 * xv))
            grows[e, pl.ds(0, 16)] = g0v * wv
            grows[e, pl.ds(16, 16)] = g1v * wv
            return 0

          lax.fori_loop(0, CHB, estep, 0, unroll=4)
          for j in range(IDX_GB):
            pltpu.sync_copy(grows.at[pl.ds(j * 128, 128)],
                            out_acc.at[idx_d.at[k * IDX_GB + j]], add=True)

          @pl.when(q < npair - 1)
          def _():
            fire(k + 2, b)
        return 0

      lax.fori_loop(0, npair, pair, 0)
      return 0

    lax.fori_loop(0, nmb, mb_body, 0)
    plsc.subcore_barrier()
    pltpu.sync_copy(out_acc.at[pl.ds(r0, rpt)],
                    outp_hbm.at[c, pl.ds(r0, rpt)])

  return kern


def _tc_project1(xp, W1, avs, avd, n_pad):
  nb = n_pad // 256

  def body(x_ref, w_ref, avs_ref, avd_ref, h_ref, ht_ref, as_ref, ad_ref,
           ws_ref):
    h = jnp.dot(x_ref[...], w_ref[...], preferred_element_type=jnp.float32)
    h_ref[...] = h
    ht_ref[...] = jnp.transpose(h.reshape(256, 16, 32), (1, 0, 2))
    a_s = jnp.dot(h, avs_ref[...], preferred_element_type=jnp.float32)
    a_d = jnp.dot(h, avd_ref[...], preferred_element_type=jnp.float32)
    as_ref[...] = a_s
    ad_ref[...] = a_d
    x2 = a_s + a_d
    ws_ref[...] = jnp.exp(jnp.maximum(x2, 0.2 * x2))

  return pl.pallas_call(
      body,
      grid=(nb,),
      in_specs=[
          pl.BlockSpec((256, 64), lambda i: (i, 0)),
          pl.BlockSpec((64, 512), lambda i: (0, 0)),
          pl.BlockSpec((512, 16), lambda i: (0, 0)),
          pl.BlockSpec((512, 16), lambda i: (0, 0)),
      ],
      out_specs=[
          pl.BlockSpec((256, 512), lambda i: (i, 0)),
          pl.BlockSpec((16, 256, 32), lambda i: (0, i, 0)),
          pl.BlockSpec((256, 16), lambda i: (i, 0)),
          pl.BlockSpec((256, 16), lambda i: (i, 0)),
          pl.BlockSpec((256, 16), lambda i: (i, 0)),
      ],
      out_shape=[
          jax.ShapeDtypeStruct((n_pad, 512), jnp.float32),
          jax.ShapeDtypeStruct((16, n_pad, 32), jnp.float32),
          jax.ShapeDtypeStruct((n_pad, 16), jnp.float32),
          jax.ShapeDtypeStruct((n_pad, 16), jnp.float32),
          jax.ShapeDtypeStruct((n_pad, 16), jnp.float32),
      ],
  )(xp, W1, avs, avd)


def _tc_epilogue1(h, outsc, wself, den_sum, e8, bias1, n_pad):
  nb = n_pad // 256

  def body(h_ref, o_ref, ws_ref, dn_ref, e8_ref, b_ref, out_ref):
    e8m = e8_ref[...]
    wrep = jnp.dot(ws_ref[...], e8m, preferred_element_type=jnp.float32)
    den = dn_ref[...] + ws_ref[...]
    drep = jnp.dot(den, e8m, preferred_element_type=jnp.float32)
    osc = jnp.transpose(o_ref[...], (1, 0, 2)).reshape(256, 512)
    num = osc + wrep * h_ref[...]
    v = num / drep + b_ref[...]
    out_ref[...] = jnp.where(v > 0, v, jnp.exp(jnp.minimum(v, 0.0)) - 1.0)

  return pl.pallas_call(
      body,
      grid=(nb,),
      in_specs=[
          pl.BlockSpec((256, 512), lambda i: (i, 0)),
          pl.BlockSpec((16, 256, 32), lambda i: (0, i, 0)),
          pl.BlockSpec((256, 16), lambda i: (i, 0)),
          pl.BlockSpec((256, 16), lambda i: (i, 0)),
          pl.BlockSpec((16, 512), lambda i: (0, 0)),
          pl.BlockSpec((1, 512), lambda i: (0, 0)),
      ],
      out_specs=pl.BlockSpec((256, 512), lambda i: (i, 0)),
      out_shape=jax.ShapeDtypeStruct((n_pad, 512), jnp.float32),
  )(h, outsc, wself, den_sum, e8, bias1)


def _tc_project2(h1, W48, onehot17, n_pad):
  nb = n_pad // 256

  def body(h_ref, w_ref, oh_ref, out_ref, gt_ref, at_ref):
    g = jnp.dot(
        h_ref[...], w_ref[...], preferred_element_type=jnp.float32
    ) + oh_ref[...]
    out_ref[...] = g
    gt_ref[...] = g[:, 0:32]
    col = lax.broadcasted_iota(jnp.int32, (256, 16), 1)
    at_ref[...] = jnp.where(col == 0, g[:, 32:33], 0.0)

  return pl.pallas_call(
      body,
      grid=(nb,),
      in_specs=[
          pl.BlockSpec((256, 512), lambda i: (i, 0)),
          pl.BlockSpec((512, 48), lambda i: (0, 0)),
          pl.BlockSpec((1, 48), lambda i: (0, 0)),
      ],
      out_specs=[
          pl.BlockSpec((256, 48), lambda i: (i, 0)),
          pl.BlockSpec((256, 32), lambda i: (i, 0)),
          pl.BlockSpec((256, 16), lambda i: (i, 0)),
      ],
      out_shape=[
          jax.ShapeDtypeStruct((n_pad, 48), jnp.float32),
          jax.ShapeDtypeStruct((n_pad, 32), jnp.float32),
          jax.ShapeDtypeStruct((n_pad, 16), jnp.float32),
      ],
  )(h1, W48, onehot17)


def _tc_epilogue2(g48, p0, p1, bias2p, n_pad):
  nb = n_pad // 256

  def body(g_ref, p0_ref, p1_ref, b_ref, out_ref):
    g = g_ref[...]
    g32 = g[:, 0:32]
    a2 = g[:, 18:19] + g[:, 32:33]
    ws2 = jnp.exp(jnp.maximum(a2, 0.2 * a2))
    num = p0_ref[...] + p1_ref[...] + ws2 * g32
    den = num[:, 17:18]
    logit = num / den + b_ref[...]
    col = lax.broadcasted_iota(jnp.int32, logit.shape, 1)
    masked = jnp.where(col < 17, logit, -1e30)
    mx = jnp.max(masked, axis=1, keepdims=True)
    lse = jnp.log(jnp.sum(jnp.exp(masked - mx), axis=1, keepdims=True)) + mx
    out_ref[...] = masked - lse

  return pl.pallas_call(
      body,
      grid=(nb,),
      in_specs=[
          pl.BlockSpec((256, 48), lambda i: (i, 0)),
          pl.BlockSpec((256, 32), lambda i: (i, 0)),
          pl.BlockSpec((256, 32), lambda i: (i, 0)),
          pl.BlockSpec((1, 32), lambda i: (0, 0)),
      ],
      out_specs=pl.BlockSpec((256, 32), lambda i: (i, 0)),
      out_shape=jax.ShapeDtypeStruct((n_pad, 32), jnp.float32),
  )(g48, p0, p1, bias2p)


def kernel(x, edge_index, W1, att_src1, att_dst1, bias1,
           W2, att_src2, att_dst2, bias2):
  n = x.shape[0]
  e = edge_index.shape[1]
  n_pad = ((n + 1 + 255) // 256) * 256
  e_pad = ((e + NW * CH - 1) // (NW * CH)) * (NW * CH)

  src = jnp.concatenate(
      [edge_index[0], jnp.full((e_pad - e,), n, jnp.int32)]
  ).reshape(e_pad // 128, 128)
  dst = jnp.concatenate(
      [edge_index[1], jnp.full((e_pad - e,), n, jnp.int32)]
  ).reshape(e_pad // 128, 128)
  xp = jnp.pad(x, ((0, n_pad - n), (0, 0)))

  eye8 = jnp.eye(8, dtype=jnp.float32)
  avs = jnp.pad((eye8[:, None, :] * att_src1[:, :, None]).reshape(512, 8),
                ((0, 0), (0, 8)))
  avd = jnp.pad((eye8[:, None, :] * att_dst1[:, :, None]).reshape(512, 8),
                ((0, 0), (0, 8)))
  e8 = jnp.repeat(jnp.pad(eye8, ((0, 8), (0, 0))), 64, axis=1)
  b1 = bias1.reshape(1, 512)

  h, h_t, asrc, adst, wself = _tc_project1(xp, W1, avs, avd, n_pad)
  w_rows, den_p = _make_edge_weight_kernel(n_pad, e_pad)(
      src, dst, asrc, adst)
  outp = _make_aggregate1_kernel(n_pad, e_pad)(src, dst, h_t, w_rows)
  den_sum = den_p[0] + den_p[1]
  h1 = _tc_epilogue1(h, outp, wself, den_sum, e8, b1, n_pad)

  gs = W2 @ att_src2.reshape(17)
  gd = W2 @ att_dst2.reshape(17)
  W48 = jnp.concatenate(
      [W2, jnp.zeros((512, 1), jnp.float32), gs[:, None],
       jnp.zeros((512, 13), jnp.float32), gd[:, None],
       jnp.zeros((512, 15), jnp.float32)], axis=1)
  onehot17 = (jnp.arange(48) == 17).astype(jnp.float32).reshape(1, 48)
  g48, gtab, atab = _tc_project2(h1, W48, onehot17, n_pad)
  out2p = _make_layer2_kernel(n_pad, e_pad)(src, dst, gtab, atab)
  b2 = jnp.pad(bias2, (0, 15)).reshape(1, 32)
  lsm = _tc_epilogue2(g48, out2p[0], out2p[1], b2, n_pad)
  return lsm[:n, :17]
